# Initial kernel scaffold; baseline (speedup 1.0000x reference)
import jax
import jax.numpy as jnp
from jax.experimental import pallas as pl


def kernel(x_features, x_graph):
    raise NotImplementedError("write your pallas kernel here")


# speedup vs baseline: 2.8460x; 2.8460x over previous
import functools

import jax
import jax.numpy as jnp
from jax import lax
from jax.experimental import pallas as pl
from jax.experimental.pallas import tpu as pltpu
from jax.experimental.pallas import tpu_sc as plsc

N = 10000
CUT = 32
D = 128
NC = 2
NS = 16
NW = NC * NS

C_NODES = 4
C_EDGES = C_NODES * CUT
NUM_CHUNKS = N // C_NODES
CHUNKS_PER_W = -(-NUM_CHUNKS // NW)


def _sc_body(feat_hbm, gflat_hbm, out_hbm, idx_v, rows_v, cent_v, gsem, csem):
    wid = lax.axis_index("s") * NC + lax.axis_index("c")

    def chunk_body(t, carry):
        chunk = wid + t * NW

        @pl.when(chunk < NUM_CHUNKS)
        def _():
            n0 = chunk * C_NODES
            e0 = chunk * C_EDGES
            pltpu.sync_copy(gflat_hbm.at[pl.ds(e0, C_EDGES)], idx_v)
            gcopy = pltpu.make_async_copy(feat_hbm.at[idx_v], rows_v, gsem)
            gcopy.start()
            ccopy = pltpu.make_async_copy(
                feat_hbm.at[pl.ds(n0, C_NODES)], cent_v, csem)
            ccopy.start()
            ccopy.wait()
            gcopy.wait()
            for i in range(C_NODES):
                cvecs = [cent_v[i, pl.ds(dv * 16, 16)] for dv in range(8)]

                def edge_body(c, _, i=i, cvecs=cvecs):
                    r = i * CUT + c
                    for dv in range(8):
                        sl = pl.ds(dv * 16, 16)
                        rows_v[r, sl] = rows_v[r, sl] - cvecs[dv]
                    return 0

                lax.fori_loop(0, CUT, edge_body, 0)
            pltpu.sync_copy(rows_v, out_hbm.at[pl.ds(e0, C_EDGES)])

        return carry

    lax.fori_loop(0, CHUNKS_PER_W, chunk_body, 0)


@functools.partial(jax.jit, static_argnums=())
def _sc_expand(feat, gflat):
    mesh = plsc.VectorSubcoreMesh(core_axis_name="c", subcore_axis_name="s")
    return pl.kernel(
        _sc_body,
        mesh=mesh,
        out_type=jax.ShapeDtypeStruct((N * CUT, D), jnp.float32),
        scratch_types=[
            pltpu.VMEM((C_EDGES,), jnp.int32),
            pltpu.VMEM((C_EDGES, D), jnp.float32),
            pltpu.VMEM((C_NODES, D), jnp.float32),
            pltpu.SemaphoreType.DMA,
            pltpu.SemaphoreType.DMA,
        ],
    )(feat, gflat)


def kernel(x_features, x_graph):
    feat = x_features.reshape(N, D)
    gflat = x_graph.astype(jnp.int32).reshape(N * CUT)
    out = _sc_expand(feat, gflat)
    return out.reshape(1, N, CUT, D)
